# baseline (speedup 1.0000x reference)
import jax
import jax.numpy as jnp
from jax import lax
from jax.experimental import pallas as pl
from jax.experimental.pallas import tpu as pltpu
from jax.experimental.pallas import tpu_sc as plsc

_N_EXPERTS = 8
_TOP_K = 2
_LB_WEIGHT = 0.01
_NEG_BIG = -1e30


def _logits_body(x_ref, gw_ref, lt_ref, pacc_ref):
    i = pl.program_id(0)
    l = lax.dot_general(
        gw_ref[...], x_ref[...], (((1,), (1,)), ((), ())),
        preferred_element_type=jnp.float32)
    lt_ref[...] = l
    m1 = jnp.max(l, axis=0, keepdims=True)
    t = jnp.exp(l - m1)
    denom = jnp.sum(t, axis=0, keepdims=True)
    pc = jnp.sum(t / denom, axis=1, keepdims=True)

    @pl.when(i == 0)
    def _init():
        pacc_ref[...] = jnp.zeros_like(pacc_ref)

    pacc_ref[...] += jnp.broadcast_to(pc, pacc_ref.shape)


def _make_sc_router(n_tok):
    info = plsc.get_sparse_core_info()
    nc, ns, nl = info.num_cores, info.num_subcores, info.num_lanes
    nw = nc * ns
    tpw = n_tok // nw
    ngrp = tpw // nl
    E = _N_EXPERTS

    mesh = plsc.VectorSubcoreMesh(core_axis_name="c", subcore_axis_name="s")

    def body(lt_hbm, w_hbm, i_hbm, fp_hbm, l_v, w_v, i_v, st_v):
        wid = lax.axis_index("s") * nc + lax.axis_index("c")
        base = wid * tpw
        pltpu.sync_copy(lt_hbm.at[:, pl.ds(base, tpw)], l_v)

        def grp(g, accs):
            ls = [l_v[e, pl.ds(g * nl, nl)] for e in range(E)]
            m1 = ls[0]
            i1 = jnp.zeros((nl,), jnp.int32)
            for e in range(1, E):
                c = ls[e] > m1
                m1 = jnp.where(c, ls[e], m1)
                i1 = jnp.where(c, e, i1)
            m2 = jnp.full((nl,), _NEG_BIG, jnp.float32)
            i2 = jnp.zeros((nl,), jnp.int32)
            for e in range(E):
                c = (i1 != e) & (ls[e] > m2)
                m2 = jnp.where(c, ls[e], m2)
                i2 = jnp.where(c, e, i2)
            r = jnp.exp(m2 - m1)
            w1 = 1.0 / (1.0 + r)
            w2 = r / (1.0 + r)
            w_v[0, pl.ds(g * nl, nl)] = w1
            w_v[1, pl.ds(g * nl, nl)] = w2
            i_v[0, pl.ds(g * nl, nl)] = i1
            i_v[1, pl.ds(g * nl, nl)] = i2
            return tuple(
                accs[e] + jnp.where((i1 == e) | (i2 == e), 1.0, 0.0)
                for e in range(E))

        zero = jnp.zeros((nl,), jnp.float32)
        accs = lax.fori_loop(0, ngrp, grp, tuple([zero] * E))
        pltpu.sync_copy(w_v, w_hbm.at[:, pl.ds(base, tpw)])
        pltpu.sync_copy(i_v, i_hbm.at[:, pl.ds(base, tpw)])
        for rix in range(E):
            st_v[rix, :] = accs[rix]
        pltpu.sync_copy(st_v, fp_hbm.at[pl.ds(wid * E, E)])

    out_type = [
        jax.ShapeDtypeStruct((_TOP_K, n_tok), jnp.float32),
        jax.ShapeDtypeStruct((_TOP_K, n_tok), jnp.int32),
        jax.ShapeDtypeStruct((nw * E, nl), jnp.float32),
    ]
    scratch_types = [
        pltpu.VMEM((E, tpw), jnp.float32),
        pltpu.VMEM((_TOP_K, tpw), jnp.float32),
        pltpu.VMEM((_TOP_K, tpw), jnp.int32),
        pltpu.VMEM((E, nl), jnp.float32),
    ]
    return pl.kernel(body, mesh=mesh, out_type=out_type,
                     scratch_types=scratch_types), nw


def _make_aux_body(n_tok, nrow, nl):
    E = _N_EXPERTS

    def aux_body(fp_ref, pacc_ref, aux_ref):
        a = fp_ref[...]
        rmod = lax.broadcasted_iota(jnp.int32, (nrow, nl), 0) % E
        s = jnp.float32(0.0)
        for e in range(E):
            fs = jnp.sum(jnp.where(rmod == e, a, 0.0))
            s = s + fs * pacc_ref[e, 0]
        n = jnp.float32(n_tok)
        aux_ref[...] = (E * _LB_WEIGHT * s / (n * n)).reshape(1, 1)

    return aux_body


def kernel(x, gate_w):
    b, s, d = x.shape
    n_tok = b * s
    xf = x.reshape(n_tok, d)
    blk = 2048
    grid = n_tok // blk

    lt, pacc = pl.pallas_call(
        _logits_body,
        grid=(grid,),
        in_specs=[
            pl.BlockSpec((blk, d), lambda i: (i, 0)),
            pl.BlockSpec((_N_EXPERTS, d), lambda i: (0, 0)),
        ],
        out_specs=[
            pl.BlockSpec((_N_EXPERTS, blk), lambda i: (0, i)),
            pl.BlockSpec((_N_EXPERTS, 128), lambda i: (0, 0)),
        ],
        out_shape=[
            jax.ShapeDtypeStruct((_N_EXPERTS, n_tok), jnp.float32),
            jax.ShapeDtypeStruct((_N_EXPERTS, 128), jnp.float32),
        ],
    )(xf, gate_w)

    sc_router, nw = _make_sc_router(n_tok)
    w_t, i_t, fp = sc_router(lt)

    nrow = nw * _N_EXPERTS
    nl = fp.shape[1]
    aux = pl.pallas_call(
        _make_aux_body(n_tok, nrow, nl),
        out_shape=jax.ShapeDtypeStruct((1, 1), jnp.float32),
    )(fp, pacc)

    top_k_weights = w_t.T.reshape(b, s, _TOP_K)
    top_k_indices = i_t.T.reshape(b, s, _TOP_K)
    return (top_k_weights, top_k_indices, aux[0, 0])


# speedup vs baseline: 1.0084x; 1.0084x over previous
import jax
import jax.numpy as jnp
from jax import lax
from jax.experimental import pallas as pl
from jax.experimental.pallas import tpu as pltpu
from jax.experimental.pallas import tpu_sc as plsc

_N_EXPERTS = 8
_TOP_K = 2
_LB_WEIGHT = 0.01
_NEG_BIG = -1e30


def _logits_body(x_ref, gw_ref, lt_ref, pacc_ref):
    i = pl.program_id(0)
    l = lax.dot_general(
        gw_ref[...], x_ref[...], (((1,), (1,)), ((), ())),
        preferred_element_type=jnp.float32)
    lt_ref[...] = l
    m1 = jnp.max(l, axis=0, keepdims=True)
    t = jnp.exp(l - m1)
    denom = jnp.sum(t, axis=0, keepdims=True)
    pc = jnp.sum(t / denom, axis=1, keepdims=True)

    @pl.when(i == 0)
    def _init():
        pacc_ref[...] = jnp.zeros_like(pacc_ref)

    pacc_ref[...] += jnp.broadcast_to(pc, pacc_ref.shape)


def _make_sc_router(n_tok):
    info = plsc.get_sparse_core_info()
    nc, ns, nl = info.num_cores, info.num_subcores, info.num_lanes
    nw = nc * ns
    tpw = n_tok // nw
    ngrp = tpw // nl
    E = _N_EXPERTS

    mesh = plsc.VectorSubcoreMesh(core_axis_name="c", subcore_axis_name="s")

    def body(lt_hbm, w_hbm, i_hbm, fp_hbm, l_v, w_v, i_v, st_v):
        wid = lax.axis_index("s") * nc + lax.axis_index("c")
        base = wid * tpw
        pltpu.sync_copy(lt_hbm.at[:, pl.ds(base, tpw)], l_v)

        def grp(g, accs):
            ls = [l_v[e, pl.ds(g * nl, nl)] for e in range(E)]
            m1 = ls[0]
            i1 = jnp.zeros((nl,), jnp.int32)
            for e in range(1, E):
                c = ls[e] > m1
                m1 = jnp.where(c, ls[e], m1)
                i1 = jnp.where(c, e, i1)
            m2 = jnp.full((nl,), _NEG_BIG, jnp.float32)
            i2 = jnp.zeros((nl,), jnp.int32)
            for e in range(E):
                c = (i1 != e) & (ls[e] > m2)
                m2 = jnp.where(c, ls[e], m2)
                i2 = jnp.where(c, e, i2)
            r = jnp.exp(m2 - m1)
            w1 = 1.0 / (1.0 + r)
            w2 = r / (1.0 + r)
            w_v[0, pl.ds(g * nl, nl)] = w1
            w_v[1, pl.ds(g * nl, nl)] = w2
            i_v[0, pl.ds(g * nl, nl)] = i1
            i_v[1, pl.ds(g * nl, nl)] = i2
            return tuple(
                accs[e] + jnp.where((i1 == e) | (i2 == e), 1.0, 0.0)
                for e in range(E))

        zero = jnp.zeros((nl,), jnp.float32)
        accs = lax.fori_loop(0, ngrp, grp, tuple([zero] * E))
        pltpu.sync_copy(w_v, w_hbm.at[:, pl.ds(base, tpw)])
        pltpu.sync_copy(i_v, i_hbm.at[:, pl.ds(base, tpw)])
        for rix in range(E):
            st_v[rix, :] = accs[rix]
        pltpu.sync_copy(st_v, fp_hbm.at[pl.ds(wid * E, E)])

    out_type = [
        jax.ShapeDtypeStruct((_TOP_K, n_tok), jnp.float32),
        jax.ShapeDtypeStruct((_TOP_K, n_tok), jnp.int32),
        jax.ShapeDtypeStruct((nw * E, nl), jnp.float32),
    ]
    scratch_types = [
        pltpu.VMEM((E, tpw), jnp.float32),
        pltpu.VMEM((_TOP_K, tpw), jnp.float32),
        pltpu.VMEM((_TOP_K, tpw), jnp.int32),
        pltpu.VMEM((E, nl), jnp.float32),
    ]
    return pl.kernel(body, mesh=mesh, out_type=out_type,
                     scratch_types=scratch_types), nw


def _make_aux_body(n_tok, nrow, nl):
    E = _N_EXPERTS

    def aux_body(fp_ref, pacc_ref, aux_ref):
        a = fp_ref[...]
        rmod = lax.broadcasted_iota(jnp.int32, (nrow, nl), 0) % E
        s = jnp.float32(0.0)
        for e in range(E):
            fs = jnp.sum(jnp.where(rmod == e, a, 0.0))
            s = s + fs * pacc_ref[e, 0]
        n = jnp.float32(n_tok)
        aux_ref[...] = (E * _LB_WEIGHT * s / (n * n)).reshape(1, 1)

    return aux_body


def kernel(x, gate_w):
    b, s, d = x.shape
    n_tok = b * s
    xf = x.reshape(n_tok, d)
    blk = 1024
    grid = n_tok // blk

    lt, pacc = pl.pallas_call(
        _logits_body,
        grid=(grid,),
        in_specs=[
            pl.BlockSpec((blk, d), lambda i: (i, 0)),
            pl.BlockSpec((_N_EXPERTS, d), lambda i: (0, 0)),
        ],
        out_specs=[
            pl.BlockSpec((_N_EXPERTS, blk), lambda i: (0, i)),
            pl.BlockSpec((_N_EXPERTS, 128), lambda i: (0, 0)),
        ],
        out_shape=[
            jax.ShapeDtypeStruct((_N_EXPERTS, n_tok), jnp.float32),
            jax.ShapeDtypeStruct((_N_EXPERTS, 128), jnp.float32),
        ],
    )(xf, gate_w)

    sc_router, nw = _make_sc_router(n_tok)
    w_t, i_t, fp = sc_router(lt)

    nrow = nw * _N_EXPERTS
    nl = fp.shape[1]
    aux = pl.pallas_call(
        _make_aux_body(n_tok, nrow, nl),
        out_shape=jax.ShapeDtypeStruct((1, 1), jnp.float32),
    )(fp, pacc)

    top_k_weights = w_t.T.reshape(b, s, _TOP_K)
    top_k_indices = i_t.T.reshape(b, s, _TOP_K)
    return (top_k_weights, top_k_indices, aux[0, 0])


# speedup vs baseline: 1.0132x; 1.0048x over previous
import jax
import jax.numpy as jnp
from jax import lax
from jax.experimental import pallas as pl
from jax.experimental.pallas import tpu as pltpu
from jax.experimental.pallas import tpu_sc as plsc

_N_EXPERTS = 8
_TOP_K = 2
_LB_WEIGHT = 0.01
_NEG_BIG = -1e30


def _lane_gather(v, idx):
    return lax.gather(
        v, idx[:, None],
        lax.GatherDimensionNumbers(
            offset_dims=(), collapsed_slice_dims=(0,), start_index_map=(0,)),
        (1,), mode=lax.GatherScatterMode.PROMISE_IN_BOUNDS)


def _logits_body(x_ref, gw_ref, lt_ref):
    lt_ref[...] = lax.dot_general(
        gw_ref[...], x_ref[...], (((1,), (1,)), ((), ())),
        preferred_element_type=jnp.float32)


def _make_sc_router(n_tok):
    info = plsc.get_sparse_core_info()
    nc, ns, nl = info.num_cores, info.num_subcores, info.num_lanes
    nw = nc * ns
    tpw = n_tok // nw
    ngrp = tpw // nl
    E = _N_EXPERTS

    mesh = plsc.VectorSubcoreMesh(core_axis_name="c", subcore_axis_name="s")

    def body(lt_hbm, w_hbm, i_hbm, fp_hbm, l_v, w_v, i_v, st_v):
        wid = lax.axis_index("s") * nc + lax.axis_index("c")
        base = wid * tpw
        pltpu.sync_copy(lt_hbm.at[:, pl.ds(base, tpw)], l_v)
        lane = lax.iota(jnp.int32, nl)

        def grp(g, accs):
            ls = [l_v[e, pl.ds(g * nl, nl)] for e in range(E)]
            m1 = ls[0]
            i1 = jnp.zeros((nl,), jnp.int32)
            for e in range(1, E):
                c = ls[e] > m1
                m1 = jnp.where(c, ls[e], m1)
                i1 = jnp.where(c, e, i1)
            m2 = jnp.full((nl,), _NEG_BIG, jnp.float32)
            i2 = jnp.zeros((nl,), jnp.int32)
            for e in range(E):
                c = (i1 != e) & (ls[e] > m2)
                m2 = jnp.where(c, ls[e], m2)
                i2 = jnp.where(c, e, i2)
            r = jnp.exp(m2 - m1)
            w1 = 1.0 / (1.0 + r)
            w2 = r / (1.0 + r)
            w_v[0, pl.ds(g * nl, nl)] = w1
            w_v[1, pl.ds(g * nl, nl)] = w2
            i_v[0, pl.ds(g * nl, nl)] = i1
            i_v[1, pl.ds(g * nl, nl)] = i2
            ts = [jnp.exp(ls[e] - m1) for e in range(E)]
            denom = ts[0]
            for e in range(1, E):
                denom = denom + ts[e]
            inv = 1.0 / denom
            out = []
            for e in range(E):
                out.append(accs[e] + jnp.where((i1 == e) | (i2 == e), 1.0, 0.0))
            for e in range(E):
                out.append(accs[E + e] + ts[e] * inv)
            return tuple(out)

        zero = jnp.zeros((nl,), jnp.float32)
        accs = lax.fori_loop(0, ngrp, grp, tuple([zero] * (2 * E)))
        pltpu.sync_copy(w_v, w_hbm.at[:, pl.ds(base, tpw)])
        pltpu.sync_copy(i_v, i_hbm.at[:, pl.ds(base, tpw)])
        for rix in range(2 * E):
            st_v[rix, :] = accs[rix]
        pltpu.sync_copy(st_v, fp_hbm.at[pl.ds(wid * 2 * E, 2 * E)])

    out_type = [
        jax.ShapeDtypeStruct((_TOP_K, n_tok), jnp.float32),
        jax.ShapeDtypeStruct((_TOP_K, n_tok), jnp.int32),
        jax.ShapeDtypeStruct((nw * 2 * E, nl), jnp.float32),
    ]
    scratch_types = [
        pltpu.VMEM((E, tpw), jnp.float32),
        pltpu.VMEM((_TOP_K, tpw), jnp.float32),
        pltpu.VMEM((_TOP_K, tpw), jnp.int32),
        pltpu.VMEM((2 * E, nl), jnp.float32),
    ]
    return pl.kernel(body, mesh=mesh, out_type=out_type,
                     scratch_types=scratch_types), nw


def _make_aux_body(n_tok, nrow, nl):
    E = _N_EXPERTS

    def aux_body(fp_ref, aux_ref):
        a = fp_ref[...]
        rmod = lax.broadcasted_iota(jnp.int32, (nrow, nl), 0) % (2 * E)
        s = jnp.float32(0.0)
        for e in range(E):
            fs = jnp.sum(jnp.where(rmod == e, a, 0.0))
            ps = jnp.sum(jnp.where(rmod == E + e, a, 0.0))
            s = s + fs * ps
        n = jnp.float32(n_tok)
        aux_ref[...] = (E * _LB_WEIGHT * s / (n * n)).reshape(1, 1)

    return aux_body


def kernel(x, gate_w):
    b, s, d = x.shape
    n_tok = b * s
    xf = x.reshape(n_tok, d)
    blk = 1024
    grid = n_tok // blk

    lt = pl.pallas_call(
        _logits_body,
        grid=(grid,),
        in_specs=[
            pl.BlockSpec((blk, d), lambda i: (i, 0)),
            pl.BlockSpec((_N_EXPERTS, d), lambda i: (0, 0)),
        ],
        out_specs=pl.BlockSpec((_N_EXPERTS, blk), lambda i: (0, i)),
        out_shape=jax.ShapeDtypeStruct((_N_EXPERTS, n_tok), jnp.float32),
    )(xf, gate_w)

    sc_router, nw = _make_sc_router(n_tok)
    w_flat, i_flat, fp = sc_router(lt)

    nrow = nw * 2 * _N_EXPERTS
    nl = fp.shape[1]
    aux = pl.pallas_call(
        _make_aux_body(n_tok, nrow, nl),
        out_shape=jax.ShapeDtypeStruct((1, 1), jnp.float32),
    )(fp)

    top_k_weights = w_flat.T.reshape(b, s, _TOP_K)
    top_k_indices = i_flat.T.reshape(b, s, _TOP_K)
    return (top_k_weights, top_k_indices, aux[0, 0])


# speedup vs baseline: 1.0159x; 1.0026x over previous
import jax
import jax.numpy as jnp
from jax import lax
from jax.experimental import pallas as pl
from jax.experimental.pallas import tpu as pltpu
from jax.experimental.pallas import tpu_sc as plsc

_N_EXPERTS = 8
_TOP_K = 2
_LB_WEIGHT = 0.01
_NEG_BIG = -1e30


def _logits_body(x_ref, gw_ref, lt_ref):
    lt_ref[...] = lax.dot_general(
        gw_ref[...], x_ref[...], (((1,), (1,)), ((), ())),
        preferred_element_type=jnp.float32)


def _make_sc_router(n_tok):
    info = plsc.get_sparse_core_info()
    nc, ns, nl = info.num_cores, info.num_subcores, info.num_lanes
    nw = nc * ns
    tpw = n_tok // nw
    ngrp = tpw // nl
    E = _N_EXPERTS

    mesh = plsc.VectorSubcoreMesh(core_axis_name="c", subcore_axis_name="s")

    def body(lt_hbm, w_hbm, i_hbm, fp_hbm, l_v, w_v, i_v, st_v):
        wid = lax.axis_index("s") * nc + lax.axis_index("c")
        base = wid * tpw
        pltpu.sync_copy(lt_hbm.at[:, pl.ds(base, tpw)], l_v)

        def grp(g, accs):
            ls = [l_v[e, pl.ds(g * nl, nl)] for e in range(E)]
            m1 = ls[0]
            i1 = jnp.zeros((nl,), jnp.int32)
            for e in range(1, E):
                c = ls[e] > m1
                m1 = jnp.where(c, ls[e], m1)
                i1 = jnp.where(c, e, i1)
            m2 = jnp.full((nl,), _NEG_BIG, jnp.float32)
            i2 = jnp.zeros((nl,), jnp.int32)
            for e in range(E):
                c = (i1 != e) & (ls[e] > m2)
                m2 = jnp.where(c, ls[e], m2)
                i2 = jnp.where(c, e, i2)
            r = jnp.exp(m2 - m1)
            w1 = 1.0 / (1.0 + r)
            w2 = r / (1.0 + r)
            w_v[0, pl.ds(g * nl, nl)] = w1
            w_v[1, pl.ds(g * nl, nl)] = w2
            i_v[0, pl.ds(g * nl, nl)] = i1
            i_v[1, pl.ds(g * nl, nl)] = i2
            ts = [jnp.exp(ls[e] - m1) for e in range(E)]
            denom = ts[0]
            for e in range(1, E):
                denom = denom + ts[e]
            inv = 1.0 / denom
            out = []
            for e in range(E):
                out.append(accs[e] + jnp.where((i1 == e) | (i2 == e), 1.0, 0.0))
            for e in range(E):
                out.append(accs[E + e] + ts[e] * inv)
            return tuple(out)

        zero = jnp.zeros((nl,), jnp.float32)
        accs = lax.fori_loop(0, ngrp, grp, tuple([zero] * (2 * E)))
        pltpu.sync_copy(w_v, w_hbm.at[:, pl.ds(base, tpw)])
        pltpu.sync_copy(i_v, i_hbm.at[:, pl.ds(base, tpw)])
        for rix in range(2 * E):
            st_v[rix, :] = accs[rix]
        pltpu.sync_copy(st_v, fp_hbm.at[pl.ds(wid * 2 * E, 2 * E)])

    out_type = [
        jax.ShapeDtypeStruct((_TOP_K, n_tok), jnp.float32),
        jax.ShapeDtypeStruct((_TOP_K, n_tok), jnp.int32),
        jax.ShapeDtypeStruct((nw * 2 * E, nl), jnp.float32),
    ]
    scratch_types = [
        pltpu.VMEM((E, tpw), jnp.float32),
        pltpu.VMEM((_TOP_K, tpw), jnp.float32),
        pltpu.VMEM((_TOP_K, tpw), jnp.int32),
        pltpu.VMEM((2 * E, nl), jnp.float32),
    ]
    return pl.kernel(body, mesh=mesh, out_type=out_type,
                     scratch_types=scratch_types), nw


def _make_aux_body(n_tok, nrow, nl):
    E = _N_EXPERTS

    def aux_body(fp_ref, aux_ref):
        a = fp_ref[...]
        rmod = lax.broadcasted_iota(jnp.int32, (nrow, nl), 0) % (2 * E)
        s = jnp.float32(0.0)
        for e in range(E):
            fs = jnp.sum(jnp.where(rmod == e, a, 0.0))
            ps = jnp.sum(jnp.where(rmod == E + e, a, 0.0))
            s = s + fs * ps
        n = jnp.float32(n_tok)
        aux_ref[...] = (E * _LB_WEIGHT * s / (n * n)).reshape(1, 1)

    return aux_body


def kernel(x, gate_w):
    b, s, d = x.shape
    n_tok = b * s
    xf = x.reshape(n_tok, d)
    blk = 1024
    grid = n_tok // blk

    lt = pl.pallas_call(
        _logits_body,
        grid=(grid,),
        in_specs=[
            pl.BlockSpec((blk, d), lambda i: (i, 0)),
            pl.BlockSpec((_N_EXPERTS, d), lambda i: (0, 0)),
        ],
        out_specs=pl.BlockSpec((_N_EXPERTS, blk), lambda i: (0, i)),
        out_shape=jax.ShapeDtypeStruct((_N_EXPERTS, n_tok), jnp.float32),
    )(xf, gate_w)

    sc_router, nw = _make_sc_router(n_tok)
    w_flat, i_flat, fp = sc_router(lt)

    nrow = nw * 2 * _N_EXPERTS
    nl = fp.shape[1]
    aux = pl.pallas_call(
        _make_aux_body(n_tok, nrow, nl),
        out_shape=jax.ShapeDtypeStruct((1, 1), jnp.float32),
    )(fp)

    top_k_weights = w_flat.T.reshape(b, s, _TOP_K)
    top_k_indices = i_flat.T.reshape(b, s, _TOP_K)
    return (top_k_weights, top_k_indices, aux[0, 0])
